# baseline (speedup 1.0000x reference)
import functools

import jax
import jax.numpy as jnp
from jax import lax
from jax.experimental import pallas as pl
from jax.experimental.pallas import tpu as pltpu
from jax.experimental.pallas import tpu_sc as plsc

NC = 2
NS = 16
NW = NC * NS
CHUNK = 128
BLK = 1024


def _cdiv(a, b):
  return (a + b - 1) // b


def _deg_body(n_pad, cpt, dst_hbm, zeros_hbm, parts_hbm, didx_v, ones_v, dacc):
  cid = lax.axis_index("c")
  sid = lax.axis_index("s")
  wid = sid * NC + cid
  rpt = n_pad // NS
  pltpu.sync_copy(zeros_hbm.at[pl.ds(sid * rpt, rpt)],
                  dacc.at[pl.ds(sid * rpt, rpt)])
  pltpu.sync_copy(dst_hbm.at[wid], didx_v)
  for j in range(CHUNK // 16):
    ones_v[pl.ds(j * 16, 16)] = jnp.ones((16,), jnp.float32)
  plsc.subcore_barrier()

  def chunk(i, carry):
    pltpu.sync_copy(ones_v, dacc.at[didx_v.at[i]], add=True)
    return carry

  lax.fori_loop(0, cpt, chunk, 0)
  plsc.subcore_barrier()
  pltpu.sync_copy(dacc.at[pl.ds(sid * rpt, rpt)],
                  parts_hbm.at[cid, pl.ds(sid * rpt, rpt)])


def _unpack_src(packed_v, i, sref):
  for k in range(CHUNK // 16):
    w = packed_v[i, pl.ds(k * 16, 16)]
    sref[pl.ds(k * 16, 16)] = jnp.bitwise_and(w, 0xFFFF)


def _unpack_dst(packed_v, i, dref):
  for k in range(CHUNK // 16):
    w = packed_v[i, pl.ds(k * 16, 16)]
    dref[pl.ds(k * 16, 16)] = lax.shift_right_logical(w, 16)


def _agg_body(n_pad, d, q0, q1, y_hbm, pk_hbm, parts_hbm,
              pk_v, sidx0, sidx1, didx_c, rows0, rows1, acc, g0, g1):
  cid = lax.axis_index("c")
  sid = lax.axis_index("s")
  rpt = n_pad // NS
  is0 = cid == 0
  with jax.named_scope("agg_zero"):
    def zrow(i, c):
      for k in range(d // 16):
        rows0[i, pl.ds(k * 16, 16)] = jnp.zeros((16,), jnp.float32)
      return c

    lax.fori_loop(0, CHUNK, zrow, 0)

    def zcopy(i, c):
      pltpu.sync_copy(rows0, acc.at[pl.ds(sid * rpt + i * CHUNK, CHUNK)])
      return c

    lax.fori_loop(0, rpt // CHUNK, zcopy, 0)
    plsc.subcore_barrier()

  with jax.named_scope("agg_loop"):
    @pl.when(is0)
    def _():
      pltpu.sync_copy(pk_hbm.at[pl.ds(sid * q0, q0)], pk_v.at[pl.ds(0, q0)])
      _unpack_src(pk_v, 0, sidx0)
      pltpu.async_copy(y_hbm.at[sidx0], rows0, g0)
      nj = q0 // 2

      def body(j, carry):
        i0 = 2 * j
        _unpack_src(pk_v, i0 + 1, sidx1)
        pltpu.async_copy(y_hbm.at[sidx1], rows1, g1)
        pltpu.make_async_copy(y_hbm.at[sidx0], rows0, g0).wait()
        _unpack_dst(pk_v, i0, didx_c)
        pltpu.sync_copy(rows0, acc.at[didx_c], add=True)

        @pl.when(j + 1 < nj)
        def _():
          _unpack_src(pk_v, i0 + 2, sidx0)
          pltpu.async_copy(y_hbm.at[sidx0], rows0, g0)

        pltpu.make_async_copy(y_hbm.at[sidx1], rows1, g1).wait()
        _unpack_dst(pk_v, i0 + 1, didx_c)
        pltpu.sync_copy(rows1, acc.at[didx_c], add=True)
        return carry

      lax.fori_loop(0, nj, body, 0)

    @pl.when(jnp.logical_not(is0))
    def _():
      pltpu.sync_copy(pk_hbm.at[pl.ds(NS * q0 + sid * q1, q1)],
                      pk_v.at[pl.ds(0, q1)])

      def chunk(i, carry):
        _unpack_src(pk_v, i, sidx0)
        pltpu.async_copy(y_hbm.at[sidx0], rows0, g0).wait()
        _unpack_dst(pk_v, i, didx_c)
        pltpu.sync_copy(rows0, acc.at[didx_c], add=True)
        return carry

      lax.fori_loop(0, q1, chunk, 0)

    plsc.subcore_barrier()
  with jax.named_scope("agg_wb"):
    pltpu.sync_copy(acc.at[pl.ds(sid * rpt, rpt)],
                    parts_hbm.at[cid, pl.ds(sid * rpt, rpt)])


def _sc_mesh(nc=NC):
  return plsc.VectorSubcoreMesh(core_axis_name="c", subcore_axis_name="s",
                                num_cores=nc, num_subcores=NS)


def _deg_kernel(n_pad, cpt):
  return pl.kernel(
      functools.partial(_deg_body, n_pad, cpt),
      out_type=jax.ShapeDtypeStruct((NC, n_pad), jnp.float32),
      mesh=_sc_mesh(),
      scratch_types=[
          pltpu.VMEM((cpt, CHUNK), jnp.int32),
          pltpu.VMEM((CHUNK,), jnp.float32),
          pltpu.VMEM_SHARED((n_pad,), jnp.float32),
      ],
  )


def _agg_kernel(n_pad, d, q0, q1):
  return pl.kernel(
      functools.partial(_agg_body, n_pad, d, q0, q1),
      out_type=jax.ShapeDtypeStruct((NC, n_pad, d), jnp.float32),
      mesh=_sc_mesh(),
      scratch_types=[
          pltpu.VMEM((max(q0, q1), CHUNK), jnp.int32),
          pltpu.VMEM((CHUNK,), jnp.int32),
          pltpu.VMEM((CHUNK,), jnp.int32),
          pltpu.VMEM((CHUNK,), jnp.int32),
          pltpu.VMEM((CHUNK, d), jnp.float32),
          pltpu.VMEM((CHUNK, d), jnp.float32),
          pltpu.VMEM_SHARED((n_pad, d), jnp.float32),
          pltpu.SemaphoreType.DMA,
          pltpu.SemaphoreType.DMA,
      ],
  )


def _k1_body(x_ref, w_ref, dp_ref, y_ref, dinv_ref):
  deg = dp_ref[0, :] + dp_ref[1, :] + 1.0
  dinv = lax.rsqrt(deg)
  dinv_ref[...] = dinv
  xw = jnp.dot(x_ref[...], w_ref[...], preferred_element_type=jnp.float32)
  y_ref[...] = xw * dinv[:, None]


def _k2_body(p_ref, y_ref, dinv_ref, w_ref, b_ref, a_ref, o_ref):
  dinv = dinv_ref[...][:, None]
  t = (p_ref[0] + p_ref[1] + y_ref[...]) * dinv + b_ref[...]
  z = jnp.where(t >= 0, t, a_ref[...] * t)
  zw = jnp.dot(z, w_ref[...], preferred_element_type=jnp.float32)
  o_ref[...] = zw * dinv


def _k3_body(p_ref, y_ref, dinv_ref, b_ref, a_ref, o_ref):
  dinv = dinv_ref[...][:, None]
  t = (p_ref[0] + p_ref[1] + y_ref[...]) * dinv + b_ref[...]
  o_ref[...] = jnp.where(t >= 0, t, a_ref[...] * t)


def _row_spec(d):
  return pl.BlockSpec((BLK, d), lambda i: (i, 0))


def _vec_spec():
  return pl.BlockSpec((BLK,), lambda i: (i,))


def _parts_spec(d):
  return pl.BlockSpec((NC, BLK, d), lambda i: (0, i, 0))


def _full_spec(shape, nd):
  return pl.BlockSpec(shape, lambda i: (0,) * nd)


def kernel(x, edge_index, W1, b1, a1, W2, b2, a2):
  n, d = x.shape
  e = edge_index.shape[1]
  n_pad = _cdiv(n + 1, NS * 8) * NS * 8
  n_pad = _cdiv(n_pad, BLK) * BLK
  grid = n_pad // BLK

  s_need = _cdiv(_cdiv(e, CHUNK), NS)
  q0 = max(8, 8 * round(s_need * 0.75 / 8))
  q0 = _cdiv(q0, 2) * 2
  q1 = max(8, _cdiv(s_need - q0, 8) * 8)
  totc = NS * (q0 + q1)
  e_pad = totc * CHUNK
  cpt_deg = totc // NW

  ei = edge_index.astype(jnp.int32)
  pad = jnp.full((2, e_pad - e), n, jnp.int32)
  ei = jnp.concatenate([ei, pad], axis=1)
  dst = ei[1].reshape(NW, cpt_deg, CHUNK)
  packed = (ei[0] | (ei[1] << 16)).reshape(totc, CHUNK)

  x_pad = jnp.zeros((n_pad, d), x.dtype).at[:n].set(x)
  zeros_1d = jnp.zeros((n_pad,), jnp.float32)
  b1r, a1r = b1.reshape(1, d), a1.reshape(1, d)
  b2r, a2r = b2.reshape(1, d), a2.reshape(1, d)

  dparts = _deg_kernel(n_pad, cpt_deg)(dst, zeros_1d)

  k1 = pl.pallas_call(
      _k1_body,
      grid=(grid,),
      in_specs=[_row_spec(d), _full_spec((d, d), 2),
                pl.BlockSpec((NC, BLK), lambda i: (0, i))],
      out_specs=[_row_spec(d), _vec_spec()],
      out_shape=[jax.ShapeDtypeStruct((n_pad, d), jnp.float32),
                 jax.ShapeDtypeStruct((n_pad,), jnp.float32)],
  )
  y1, dinv = k1(x_pad, W1, dparts)

  agg = _agg_kernel(n_pad, d, q0, q1)
  parts1 = agg(y1, packed)

  k2 = pl.pallas_call(
      _k2_body,
      grid=(grid,),
      in_specs=[_parts_spec(d), _row_spec(d), _vec_spec(),
                _full_spec((d, d), 2), _full_spec((1, d), 2),
                _full_spec((1, d), 2)],
      out_specs=_row_spec(d),
      out_shape=jax.ShapeDtypeStruct((n_pad, d), jnp.float32),
  )
  y2 = k2(parts1, y1, dinv, W2, b1r, a1r)

  parts2 = agg(y2, packed)

  k3 = pl.pallas_call(
      _k3_body,
      grid=(grid,),
      in_specs=[_parts_spec(d), _row_spec(d), _vec_spec(),
                _full_spec((1, d), 2), _full_spec((1, d), 2)],
      out_specs=_row_spec(d),
      out_shape=jax.ShapeDtypeStruct((n_pad, d), jnp.float32),
  )
  z = k3(parts2, y2, dinv, b2r, a2r)
  return z[:n]


# speedup vs baseline: 1.1876x; 1.1876x over previous
import functools

import jax
import jax.numpy as jnp
from jax import lax
from jax.experimental import pallas as pl
from jax.experimental.pallas import tpu as pltpu
from jax.experimental.pallas import tpu_sc as plsc

NC = 2
NS = 16
NW = NC * NS
CHUNK = 128
BLK = 1024


def _cdiv(a, b):
  return (a + b - 1) // b


def _deg_body(n_pad, cpt, dst_hbm, zeros_hbm, parts_hbm, didx_v, ones_v, dacc):
  cid = lax.axis_index("c")
  sid = lax.axis_index("s")
  wid = sid * NC + cid
  rpt = n_pad // NS
  pltpu.sync_copy(zeros_hbm.at[pl.ds(sid * rpt, rpt)],
                  dacc.at[pl.ds(sid * rpt, rpt)])
  pltpu.sync_copy(dst_hbm.at[wid], didx_v)
  for j in range(CHUNK // 16):
    ones_v[pl.ds(j * 16, 16)] = jnp.ones((16,), jnp.float32)
  plsc.subcore_barrier()

  def chunk(i, carry):
    pltpu.sync_copy(ones_v, dacc.at[didx_v.at[i]], add=True)
    return carry

  lax.fori_loop(0, cpt, chunk, 0)
  plsc.subcore_barrier()
  pltpu.sync_copy(dacc.at[pl.ds(sid * rpt, rpt)],
                  parts_hbm.at[cid, pl.ds(sid * rpt, rpt)])


def _unpack_src(packed_v, i, sref):
  for k in range(CHUNK // 16):
    w = packed_v[i, pl.ds(k * 16, 16)]
    sref[pl.ds(k * 16, 16)] = jnp.bitwise_and(w, 0xFFFF)


def _unpack_dst(packed_v, i, dref):
  for k in range(CHUNK // 16):
    w = packed_v[i, pl.ds(k * 16, 16)]
    dref[pl.ds(k * 16, 16)] = lax.shift_right_logical(w, 16)


def _agg_body(n_pad, d, cpt, y_hbm, src_hbm, dst_hbm, parts_hbm,
              sidx_v, didx_v, rows, acc, g0):
  cid = lax.axis_index("c")
  sid = lax.axis_index("s")
  wid = sid * NC + cid
  rpt = n_pad // NS
  with jax.named_scope("agg_zero"):
    def zrow(i, c):
      for k in range(d // 16):
        rows[i, pl.ds(k * 16, 16)] = jnp.zeros((16,), rows.dtype)
      return c

    lax.fori_loop(0, CHUNK, zrow, 0)

    def zcopy(i, c):
      pltpu.sync_copy(rows, acc.at[pl.ds(sid * rpt + i * CHUNK, CHUNK)])
      return c

    lax.fori_loop(0, rpt // CHUNK, zcopy, 0)
    pltpu.sync_copy(src_hbm.at[wid], sidx_v)
    pltpu.sync_copy(dst_hbm.at[wid], didx_v)
    plsc.subcore_barrier()

  with jax.named_scope("agg_loop"):
    def chunk(i, carry):
      pltpu.async_copy(y_hbm.at[sidx_v.at[i]], rows, g0).wait()
      pltpu.sync_copy(rows, acc.at[didx_v.at[i]], add=True)
      return carry

    lax.fori_loop(0, cpt, chunk, 0)
    plsc.subcore_barrier()
  with jax.named_scope("agg_wb"):
    pltpu.sync_copy(acc.at[pl.ds(sid * rpt, rpt)],
                    parts_hbm.at[cid, pl.ds(sid * rpt, rpt)])


def _sc_mesh(nc=NC):
  return plsc.VectorSubcoreMesh(core_axis_name="c", subcore_axis_name="s",
                                num_cores=nc, num_subcores=NS)


def _deg_kernel(n_pad, cpt):
  return pl.kernel(
      functools.partial(_deg_body, n_pad, cpt),
      out_type=jax.ShapeDtypeStruct((NC, n_pad), jnp.float32),
      mesh=_sc_mesh(),
      scratch_types=[
          pltpu.VMEM((cpt, CHUNK), jnp.int32),
          pltpu.VMEM((CHUNK,), jnp.float32),
          pltpu.VMEM_SHARED((n_pad,), jnp.float32),
      ],
  )


def _agg_kernel(n_pad, d, cpt, dtype):
  return pl.kernel(
      functools.partial(_agg_body, n_pad, d, cpt),
      out_type=jax.ShapeDtypeStruct((NC, n_pad, d), dtype),
      mesh=_sc_mesh(),
      scratch_types=[
          pltpu.VMEM((cpt, CHUNK), jnp.int32),
          pltpu.VMEM((cpt, CHUNK), jnp.int32),
          pltpu.VMEM((CHUNK, d), dtype),
          pltpu.VMEM_SHARED((n_pad, d), dtype),
          pltpu.SemaphoreType.DMA,
      ],
  )


def _k1_body(x_ref, w_ref, dp_ref, y_ref, dinv_ref):
  deg = dp_ref[0, :] + dp_ref[1, :] + 1.0
  dinv = lax.rsqrt(deg)
  dinv_ref[...] = dinv
  xw = jnp.dot(x_ref[...], w_ref[...], preferred_element_type=jnp.float32)
  y_ref[...] = xw * dinv[:, None]


def _k2_body(p_ref, y_ref, dinv_ref, w_ref, b_ref, a_ref, o_ref):
  dinv = dinv_ref[...][:, None]
  t = (p_ref[0] + p_ref[1] + y_ref[...]) * dinv + b_ref[...]
  z = jnp.where(t >= 0, t, a_ref[...] * t)
  zw = jnp.dot(z, w_ref[...], preferred_element_type=jnp.float32)
  o_ref[...] = zw * dinv


def _k3_body(p_ref, y_ref, dinv_ref, b_ref, a_ref, o_ref):
  dinv = dinv_ref[...][:, None]
  t = (p_ref[0] + p_ref[1] + y_ref[...]) * dinv + b_ref[...]
  o_ref[...] = jnp.where(t >= 0, t, a_ref[...] * t)


def _row_spec(d):
  return pl.BlockSpec((BLK, d), lambda i: (i, 0))


def _vec_spec():
  return pl.BlockSpec((BLK,), lambda i: (i,))


def _parts_spec(d):
  return pl.BlockSpec((NC, BLK, d), lambda i: (0, i, 0))


def _full_spec(shape, nd):
  return pl.BlockSpec(shape, lambda i: (0,) * nd)


def kernel(x, edge_index, W1, b1, a1, W2, b2, a2):
  n, d = x.shape
  e = edge_index.shape[1]
  n_pad = _cdiv(n + 1, NS * 8) * NS * 8
  n_pad = _cdiv(n_pad, BLK) * BLK
  grid = n_pad // BLK

  cpt = _cdiv(e, NW * CHUNK)
  totc = NW * cpt
  e_pad = totc * CHUNK

  ei = edge_index.astype(jnp.int32)
  pad = jnp.full((2, e_pad - e), n, jnp.int32)
  ei = jnp.concatenate([ei, pad], axis=1)
  src = ei[0].reshape(NW, cpt, CHUNK)
  dst = ei[1].reshape(NW, cpt, CHUNK)

  x_pad = jnp.zeros((n_pad, d), x.dtype).at[:n].set(x)
  zeros_1d = jnp.zeros((n_pad,), jnp.float32)
  b1r, a1r = b1.reshape(1, d), a1.reshape(1, d)
  b2r, a2r = b2.reshape(1, d), a2.reshape(1, d)

  dparts = _deg_kernel(n_pad, cpt)(dst, zeros_1d)

  k1 = pl.pallas_call(
      _k1_body,
      grid=(grid,),
      in_specs=[_row_spec(d), _full_spec((d, d), 2),
                pl.BlockSpec((NC, BLK), lambda i: (0, i))],
      out_specs=[_row_spec(d), _vec_spec()],
      out_shape=[jax.ShapeDtypeStruct((n_pad, d), jnp.float32),
                 jax.ShapeDtypeStruct((n_pad,), jnp.float32)],
  )
  y1, dinv = k1(x_pad, W1, dparts)

  agg = _agg_kernel(n_pad, d, cpt, jnp.float32)
  parts1 = agg(y1, src, dst)

  k2 = pl.pallas_call(
      _k2_body,
      grid=(grid,),
      in_specs=[_parts_spec(d), _row_spec(d), _vec_spec(),
                _full_spec((d, d), 2), _full_spec((1, d), 2),
                _full_spec((1, d), 2)],
      out_specs=_row_spec(d),
      out_shape=jax.ShapeDtypeStruct((n_pad, d), jnp.float32),
  )
  y2 = k2(parts1, y1, dinv, W2, b1r, a1r)

  parts2 = agg(y2, src, dst)

  k3 = pl.pallas_call(
      _k3_body,
      grid=(grid,),
      in_specs=[_parts_spec(d), _row_spec(d), _vec_spec(),
                _full_spec((1, d), 2), _full_spec((1, d), 2)],
      out_specs=_row_spec(d),
      out_shape=jax.ShapeDtypeStruct((n_pad, d), jnp.float32),
  )
  z = k3(parts2, y2, dinv, b2r, a2r)
  return z[:n]


# speedup vs baseline: 1.2753x; 1.0738x over previous
import functools

import jax
import jax.numpy as jnp
from jax import lax
from jax.experimental import pallas as pl
from jax.experimental.pallas import tpu as pltpu
from jax.experimental.pallas import tpu_sc as plsc

NC = 2
NS = 16
NW = NC * NS
CHUNK = 128
BLK = 1024


def _cdiv(a, b):
  return (a + b - 1) // b


def _deg_body(n_pad, cpt, dst_hbm, zeros_hbm, parts_hbm, didx_v, ones_v, dacc):
  cid = lax.axis_index("c")
  sid = lax.axis_index("s")
  wid = sid * NC + cid
  rpt = n_pad // NS
  pltpu.sync_copy(zeros_hbm.at[pl.ds(sid * rpt, rpt)],
                  dacc.at[pl.ds(sid * rpt, rpt)])
  pltpu.sync_copy(dst_hbm.at[wid], didx_v)
  for j in range(CHUNK // 16):
    ones_v[pl.ds(j * 16, 16)] = jnp.ones((16,), jnp.float32)
  plsc.subcore_barrier()

  def chunk(i, carry):
    pltpu.sync_copy(ones_v, dacc.at[didx_v.at[i]], add=True)
    return carry

  lax.fori_loop(0, cpt, chunk, 0)
  plsc.subcore_barrier()
  pltpu.sync_copy(dacc.at[pl.ds(sid * rpt, rpt)],
                  parts_hbm.at[cid, pl.ds(sid * rpt, rpt)])


def _unpack_src(packed_v, i, sref):
  for k in range(CHUNK // 16):
    w = packed_v[i, pl.ds(k * 16, 16)]
    sref[pl.ds(k * 16, 16)] = jnp.bitwise_and(w, 0xFFFF)


def _unpack_dst(packed_v, i, dref):
  for k in range(CHUNK // 16):
    w = packed_v[i, pl.ds(k * 16, 16)]
    dref[pl.ds(k * 16, 16)] = lax.shift_right_logical(w, 16)


def _agg_body(n_pad, d, cpt, y_hbm, src_hbm, dst_hbm, zeros_hbm, parts_hbm,
              sidx_v, didx_v, rows, acc, g0):
  cid = lax.axis_index("c")
  sid = lax.axis_index("s")
  wid = sid * NC + cid
  rpt = n_pad // NS
  with jax.named_scope("agg_zero"):
    pltpu.sync_copy(zeros_hbm, acc.at[pl.ds(sid * rpt, rpt)])
    pltpu.sync_copy(src_hbm.at[wid], sidx_v)
    pltpu.sync_copy(dst_hbm.at[wid], didx_v)
    plsc.subcore_barrier()

  with jax.named_scope("agg_loop"):
    def chunk(i, carry):
      pltpu.async_copy(y_hbm.at[sidx_v.at[i]], rows, g0).wait()
      pltpu.sync_copy(rows, acc.at[didx_v.at[i]], add=True)
      return carry

    lax.fori_loop(0, cpt, chunk, 0)
    plsc.subcore_barrier()
  with jax.named_scope("agg_wb"):
    pltpu.sync_copy(acc.at[pl.ds(sid * rpt, rpt)],
                    parts_hbm.at[cid, pl.ds(sid * rpt, rpt)])


def _sc_mesh(nc=NC):
  return plsc.VectorSubcoreMesh(core_axis_name="c", subcore_axis_name="s",
                                num_cores=nc, num_subcores=NS)


def _deg_kernel(n_pad, cpt):
  return pl.kernel(
      functools.partial(_deg_body, n_pad, cpt),
      out_type=jax.ShapeDtypeStruct((NC, n_pad), jnp.float32),
      mesh=_sc_mesh(),
      scratch_types=[
          pltpu.VMEM((cpt, CHUNK), jnp.int32),
          pltpu.VMEM((CHUNK,), jnp.float32),
          pltpu.VMEM_SHARED((n_pad,), jnp.float32),
      ],
  )


def _agg_kernel(n_pad, d, cpt, dtype):
  return pl.kernel(
      functools.partial(_agg_body, n_pad, d, cpt),
      out_type=jax.ShapeDtypeStruct((NC, n_pad, d), dtype),
      mesh=_sc_mesh(),
      scratch_types=[
          pltpu.VMEM((cpt, CHUNK), jnp.int32),
          pltpu.VMEM((cpt, CHUNK), jnp.int32),
          pltpu.VMEM((CHUNK, d), dtype),
          pltpu.VMEM_SHARED((n_pad, d), dtype),
          pltpu.SemaphoreType.DMA,
      ],
  )


def _k1_body(x_ref, w_ref, dp_ref, y_ref, dinv_ref):
  deg = dp_ref[0, :] + dp_ref[1, :] + 1.0
  dinv = lax.rsqrt(deg)
  dinv_ref[...] = dinv
  xw = jnp.dot(x_ref[...], w_ref[...], preferred_element_type=jnp.float32)
  y_ref[...] = xw * dinv[:, None]


def _k2_body(p_ref, y_ref, dinv_ref, w_ref, b_ref, a_ref, o_ref):
  dinv = dinv_ref[...][:, None]
  t = (p_ref[0] + p_ref[1] + y_ref[...]) * dinv + b_ref[...]
  z = jnp.where(t >= 0, t, a_ref[...] * t)
  zw = jnp.dot(z, w_ref[...], preferred_element_type=jnp.float32)
  o_ref[...] = zw * dinv


def _k3_body(p_ref, y_ref, dinv_ref, b_ref, a_ref, o_ref):
  dinv = dinv_ref[...][:, None]
  t = (p_ref[0] + p_ref[1] + y_ref[...]) * dinv + b_ref[...]
  o_ref[...] = jnp.where(t >= 0, t, a_ref[...] * t)


def _row_spec(d):
  return pl.BlockSpec((BLK, d), lambda i: (i, 0))


def _vec_spec():
  return pl.BlockSpec((BLK,), lambda i: (i,))


def _parts_spec(d):
  return pl.BlockSpec((NC, BLK, d), lambda i: (0, i, 0))


def _full_spec(shape, nd):
  return pl.BlockSpec(shape, lambda i: (0,) * nd)


def kernel(x, edge_index, W1, b1, a1, W2, b2, a2):
  n, d = x.shape
  e = edge_index.shape[1]
  n_pad = _cdiv(n + 1, NS * 8) * NS * 8
  n_pad = _cdiv(n_pad, BLK) * BLK
  grid = n_pad // BLK

  cpt = _cdiv(e, NW * CHUNK)
  totc = NW * cpt
  e_pad = totc * CHUNK

  ei = edge_index.astype(jnp.int32)
  pad = jnp.full((2, e_pad - e), n, jnp.int32)
  ei = jnp.concatenate([ei, pad], axis=1)
  src = ei[0].reshape(NW, cpt, CHUNK)
  dst = ei[1].reshape(NW, cpt, CHUNK)

  x_pad = jnp.zeros((n_pad, d), x.dtype).at[:n].set(x)
  zeros_1d = jnp.zeros((n_pad,), jnp.float32)
  b1r, a1r = b1.reshape(1, d), a1.reshape(1, d)
  b2r, a2r = b2.reshape(1, d), a2.reshape(1, d)

  dparts = _deg_kernel(n_pad, cpt)(dst, zeros_1d)

  k1 = pl.pallas_call(
      _k1_body,
      grid=(grid,),
      in_specs=[_row_spec(d), _full_spec((d, d), 2),
                pl.BlockSpec((NC, BLK), lambda i: (0, i))],
      out_specs=[_row_spec(d), _vec_spec()],
      out_shape=[jax.ShapeDtypeStruct((n_pad, d), jnp.float32),
                 jax.ShapeDtypeStruct((n_pad,), jnp.float32)],
  )
  y1, dinv = k1(x_pad, W1, dparts)

  zeros_rpt = jnp.zeros((n_pad // NS, d), jnp.float32)
  agg = _agg_kernel(n_pad, d, cpt, jnp.float32)
  parts1 = agg(y1, src, dst, zeros_rpt)

  k2 = pl.pallas_call(
      _k2_body,
      grid=(grid,),
      in_specs=[_parts_spec(d), _row_spec(d), _vec_spec(),
                _full_spec((d, d), 2), _full_spec((1, d), 2),
                _full_spec((1, d), 2)],
      out_specs=_row_spec(d),
      out_shape=jax.ShapeDtypeStruct((n_pad, d), jnp.float32),
  )
  y2 = k2(parts1, y1, dinv, W2, b1r, a1r)

  parts2 = agg(y2, src, dst, zeros_rpt)

  k3 = pl.pallas_call(
      _k3_body,
      grid=(grid,),
      in_specs=[_parts_spec(d), _row_spec(d), _vec_spec(),
                _full_spec((1, d), 2), _full_spec((1, d), 2)],
      out_specs=_row_spec(d),
      out_shape=jax.ShapeDtypeStruct((n_pad, d), jnp.float32),
  )
  z = k3(parts2, y2, dinv, b2r, a2r)
  return z[:n]
